# baseline (speedup 1.0000x reference)
import functools

import jax
import jax.numpy as jnp
from jax import lax
from jax.experimental import pallas as pl
from jax.experimental.pallas import tpu as pltpu
from jax.experimental.pallas import tpu_sc as plsc

_N_SRC = 4
_SIZE = 262144
_DELAY = 32
_BUF_LEN = _DELAY + 1

_info = plsc.get_sparse_core_info()
_NC, _NS, _NL = _info.num_cores, _info.num_subcores, _info.num_lanes
_NW = _NC * _NS
_PPS = _NW // _N_SRC
_CH = _SIZE // _PPS


def _sc_body(idx_hbm, buf_hbm, out_hbm, idx_v, chunk_v):
    wid = lax.axis_index("s") * _NC + lax.axis_index("c")
    src = wid // _PPS
    off = (wid % _PPS) * _CH
    pltpu.sync_copy(idx_hbm, idx_v)
    slot = idx_v[...][0]
    pltpu.sync_copy(buf_hbm.at[src, slot, pl.ds(off, _CH)], chunk_v)
    pltpu.sync_copy(chunk_v, out_hbm.at[src, pl.ds(off, _CH)])


_sc_slice = functools.partial(
    pl.kernel,
    out_type=jax.ShapeDtypeStruct((_N_SRC, _SIZE), jnp.float32),
    mesh=plsc.VectorSubcoreMesh(core_axis_name="c", subcore_axis_name="s"),
    scratch_types=[
        pltpu.VMEM((_NL,), jnp.int32),
        pltpu.VMEM((_CH,), jnp.float32),
    ],
    compiler_params=pltpu.CompilerParams(use_tc_tiling_on_sc=True),
)(_sc_body)


def kernel(spikes, buffer, ptr):
    del spikes
    read_idx = jnp.asarray((ptr + 1 - _DELAY) % _BUF_LEN, jnp.int32)
    idx_vec = jnp.zeros((_NL,), dtype=jnp.int32).at[0].set(read_idx)
    return _sc_slice(idx_vec, buffer)


# speedup vs baseline: 4.7852x; 4.7852x over previous
import functools

import jax
import jax.numpy as jnp
from jax import lax
from jax.experimental import pallas as pl
from jax.experimental.pallas import tpu as pltpu
from jax.experimental.pallas import tpu_sc as plsc

_N_SRC = 4
_SIZE = 262144
_DELAY = 32
_BUF_LEN = _DELAY + 1

_LANE = 128
_NCB = _SIZE // _LANE

_info = plsc.get_sparse_core_info()
_NC, _NS, _NL = _info.num_cores, _info.num_subcores, _info.num_lanes
_NW = _NC * _NS
_CBW = _NCB // _NW


def _sc_body(idx_hbm, buf_hbm, out_hbm, idx_v, chunk_v):
    wid = lax.axis_index("s") * _NC + lax.axis_index("c")
    base = wid * _CBW
    pltpu.sync_copy(idx_hbm, idx_v)
    slot = idx_v[...][0]
    pltpu.sync_copy(buf_hbm.at[slot, pl.ds(base, _CBW)], chunk_v)
    pltpu.sync_copy(chunk_v, out_hbm.at[pl.ds(base, _CBW)])


_sc_slice = functools.partial(
    pl.kernel,
    out_type=jax.ShapeDtypeStruct((_NCB, _N_SRC, _LANE), jnp.float32),
    mesh=plsc.VectorSubcoreMesh(core_axis_name="c", subcore_axis_name="s"),
    scratch_types=[
        pltpu.VMEM((_NL,), jnp.int32),
        pltpu.VMEM((_CBW, _N_SRC, _LANE), jnp.float32),
    ],
)(_sc_body)


def kernel(spikes, buffer, ptr):
    del spikes
    read_idx = jnp.asarray((ptr + 1 - _DELAY) % _BUF_LEN, jnp.int32)
    idx_vec = jnp.zeros((_NL,), dtype=jnp.int32).at[0].set(read_idx)
    buf4 = buffer.reshape(_N_SRC, _BUF_LEN, _NCB, _LANE).transpose(1, 2, 0, 3)
    out4 = _sc_slice(idx_vec, buf4)
    return out4.transpose(1, 0, 2).reshape(_N_SRC, _SIZE)
